# Initial kernel scaffold; baseline (speedup 1.0000x reference)
import jax
import jax.numpy as jnp
from jax.experimental import pallas as pl


def kernel(x, edge_index, block_instructions, lengths, emb, W_ih_f, W_hh_f, b_ih_f, b_hh_f, W_ih_b, W_hh_b, b_ih_b, b_hh_b, W_l, b_l, W_r, b_r, att, bias_gat, g_gat, be_gat, W1, b1, g1, be1, W2, b2, g2, be2, W3, b3):
    raise NotImplementedError("write your pallas kernel here")


# speedup vs baseline: 7.4457x; 7.4457x over previous
import functools

import jax
import jax.numpy as jnp
from jax import lax
from jax.experimental import pallas as pl
from jax.experimental.pallas import tpu as pltpu
from jax.experimental.pallas import tpu_sc as plsc

N = 50000
E = 800000
VOCAB = 32
EMB = 128
L = 20
HL = 32
HID = 64

BN1 = 1000


def _lstm_block(inst_ref, rinst_ref, lenf_ref, x_ref,
                tf_ref, tb_ref, whf_ref, whb_ref,
                wlx_ref, wlp_ref, bl_ref, wrx_ref, wrp_ref, br_ref,
                xl0_ref, xl1_ref, xr_ref):
    bn = inst_ref.shape[0]
    lenf = lenf_ref[...]

    def direction(iref, table_ref, wh_ref):
        h = jnp.zeros((bn, HL), jnp.float32)
        c = jnp.zeros((bn, HL), jnp.float32)
        acc = jnp.zeros((bn, HL), jnp.float32)
        table = table_ref[...]
        whT = wh_ref[...]
        iot = lax.broadcasted_iota(jnp.int32, (bn, VOCAB + 2), 1)
        for t in range(L):
            col = iref[:, t:t + 1]
            oh = (col == iot).astype(jnp.float32)
            gates = jnp.dot(oh, table, preferred_element_type=jnp.float32)
            gates = gates + jnp.dot(h, whT, preferred_element_type=jnp.float32)
            i = jax.nn.sigmoid(gates[:, 0:HL])
            f = jax.nn.sigmoid(gates[:, HL:2 * HL])
            g = jnp.tanh(gates[:, 2 * HL:3 * HL])
            o = jax.nn.sigmoid(gates[:, 3 * HL:4 * HL])
            c = f * c + i * g
            h = o * jnp.tanh(c)
            mask = (jnp.float32(t) < lenf).astype(jnp.float32)
            acc = acc + h * mask
        return acc

    accf = direction(inst_ref, tf_ref, whf_ref)
    accb = direction(rinst_ref, tb_ref, whb_ref)
    inv = 1.0 / lenf
    pf = accf * inv
    pb = accb * inv
    x = x_ref[...]
    xl = (jnp.dot(x, wlx_ref[...], preferred_element_type=jnp.float32)
          + jnp.dot(pf, wlp_ref[:HL, :], preferred_element_type=jnp.float32)
          + jnp.dot(pb, wlp_ref[HL:, :], preferred_element_type=jnp.float32)
          + bl_ref[...])
    xr = (jnp.dot(x, wrx_ref[...], preferred_element_type=jnp.float32)
          + jnp.dot(pf, wrp_ref[:HL, :], preferred_element_type=jnp.float32)
          + jnp.dot(pb, wrp_ref[HL:, :], preferred_element_type=jnp.float32)
          + br_ref[...])
    xl0_ref[...] = xl[:, 0:HL]
    xl1_ref[...] = xl[:, HL:HID]
    xr_ref[...] = xr


def _stage1(inst, rinst, lenf, x, tf, tb, whf, whb,
            wlx, wlp, bl, wrx, wrp, br):
    nblk = N // BN1
    full = lambda shape: pl.BlockSpec(shape, lambda i: (0,) * len(shape))
    blk = lambda shape: pl.BlockSpec(shape, lambda i: (i,) + (0,) * (len(shape) - 1))
    return pl.pallas_call(
        _lstm_block,
        grid=(nblk,),
        in_specs=[
            blk((BN1, L)), blk((BN1, L)), blk((BN1, 1)), blk((BN1, VOCAB)),
            full((VOCAB + 2, 4 * HL)), full((VOCAB + 2, 4 * HL)),
            full((HL, 4 * HL)), full((HL, 4 * HL)),
            full((VOCAB, HID)), full((2 * HL, HID)), full((1, HID)),
            full((VOCAB, HID)), full((2 * HL, HID)), full((1, HID)),
        ],
        out_specs=[blk((BN1, HL)), blk((BN1, HL)), blk((BN1, HID))],
        out_shape=[
            jax.ShapeDtypeStruct((N, HL), jnp.float32),
            jax.ShapeDtypeStruct((N, HL), jnp.float32),
            jax.ShapeDtypeStruct((N, HID), jnp.float32),
        ],
    )(inst, rinst, lenf, x, tf, tb, whf, whb, wlx, wlp, bl, wrx, wrp, br)


EACT = E + N
EP = 851968
KE = 128
NTILE = 32
NP2 = 51200
NP3 = 50176

_P1_CHUNKS = EP // NTILE // KE
_P2_CHUNKS = EP // 16 // KE
def _sc_mesh():
    return plsc.VectorSubcoreMesh(core_axis_name="c", subcore_axis_name="s",
                                  num_cores=2, num_subcores=16)


def _gat_pass1(src_hbm, dst_hbm, xl0_hbm, xl1_hbm, xr_hbm, att_hbm,
               exw_hbm, denp_hbm,
               sidx, didx, rl0, rl1, rr, sbuf, exbuf, attv, denloc, sem):
    wid = lax.axis_index("s") * 2 + lax.axis_index("c")
    pltpu.sync_copy(att_hbm, attv)
    iota = lax.iota(jnp.int32, 16)

    def zero_body(z, _):
        denloc[pl.ds(z * 16, 16)] = jnp.zeros((16,), jnp.float32)
        return _
    lax.fori_loop(0, NP3 // 16, zero_body, None)

    att_g = [attv[pl.ds(fg * 16, 16)] for fg in range(4)]

    def chunk_body(k, _):
        base = wid * (_P1_CHUNKS * KE) + k * KE
        pltpu.sync_copy(src_hbm.at[pl.ds(base, KE)], sidx)
        pltpu.sync_copy(dst_hbm.at[pl.ds(base, KE)], didx)
        c0 = pltpu.async_copy(xl0_hbm.at[sidx], rl0, sem)
        c1 = pltpu.async_copy(xl1_hbm.at[sidx], rl1, sem)
        c2 = pltpu.async_copy(xr_hbm.at[didx], rr, sem)
        c0.wait()
        c1.wait()
        c2.wait()

        def row_body(row, _):
            ps = jnp.zeros((16,), jnp.float32)
            for fg in range(4):
                if fg < 2:
                    a = rl0[row, pl.ds(fg * 16, 16)]
                else:
                    a = rl1[row, pl.ds((fg - 2) * 16, 16)]
                m = a + rr[row, pl.ds(fg * 16, 16)]
                m = jnp.maximum(m, m * jnp.float32(0.2))
                ps = ps + m * att_g[fg]
            sbuf[pl.ds(row * 16, 16)] = ps
            return _
        lax.fori_loop(0, KE, row_body, None, unroll=4)

        for g in range(8):
            flat = (iota + g * 16) * 16
            e = jnp.zeros((16,), jnp.float32)
            for l in range(16):
                e = e + plsc.load_gather(sbuf, [flat + l])
            ex = jnp.exp(e)
            exbuf[pl.ds(g * 16, 16)] = ex
            dv = didx[pl.ds(g * 16, 16)]
            valid = (base + g * 16 + iota) < EACT
            dv = jnp.where(valid, dv, jnp.int32(NP3 - 1))
            plsc.addupdate_scatter(denloc, [dv], ex)
        pltpu.sync_copy(exbuf, exw_hbm.at[pl.ds(base, KE)])
        return _

    lax.fori_loop(0, _P1_CHUNKS, chunk_body, None)
    pltpu.sync_copy(denloc, denp_hbm.at[pl.ds(wid * NP3, NP3)])


def _gat_pass2(src_hbm, dst_hbm, xl0_hbm, xl1_hbm, exw_hbm, denp_hbm,
               h0_hbm, h1_hbm, den_hbm,
               sidx, d2, rows, wrows, exbuf, dbuf, dsum, accum_sh, sem):
    c = lax.axis_index("c")
    sid = lax.axis_index("s")
    iota = lax.iota(jnp.int32, 16)

    def zw(row, _):
        wrows[row, pl.ds(0, 16)] = jnp.zeros((16,), jnp.float32)
        wrows[row, pl.ds(16, 16)] = jnp.zeros((16,), jnp.float32)
        return _
    lax.fori_loop(0, KE, zw, None, unroll=8)

    def zacc(cc, _):
        pltpu.sync_copy(wrows, accum_sh.at[pl.ds(sid * (NP2 // 16) + cc * KE, KE)])
        return _
    lax.fori_loop(0, NP2 // 16 // KE, zacc, None)
    plsc.subcore_barrier()

    def edge_loop(xlh_hbm):
        def chunk_body(k, _):
            base = sid * (_P2_CHUNKS * KE) + k * KE
            pltpu.sync_copy(src_hbm.at[pl.ds(base, KE)], sidx)
            pltpu.sync_copy(exw_hbm.at[pl.ds(base, KE)], exbuf)
            cg = pltpu.async_copy(xlh_hbm.at[sidx], rows, sem)
            pltpu.sync_copy(dst_hbm.at[pl.ds(base, KE)], d2)
            for g in range(8):
                dv = d2[pl.ds(g * 16, 16)]
                valid = (base + g * 16 + iota) < EACT
                d2[pl.ds(g * 16, 16)] = jnp.where(valid, dv, jnp.int32(NP2 - 1))
            cg.wait()

            def grp_body(g, _):
                exv = exbuf[pl.ds(g * 16, 16)]
                for l in range(16):
                    row = g * 16 + l
                    ex = exv[l]
                    wrows[row, pl.ds(0, 16)] = rows[row, pl.ds(0, 16)] * ex
                    wrows[row, pl.ds(16, 16)] = rows[row, pl.ds(16, 16)] * ex
                return _
            lax.fori_loop(0, KE // 16, grp_body, None)
            pltpu.sync_copy(wrows, accum_sh.at[d2], add=True)
            return _
        lax.fori_loop(0, _P2_CHUNKS, chunk_body, None)

    @pl.when(c == 0)
    def _sc0():
        edge_loop(xl0_hbm)

    @pl.when(c == 1)
    def _sc1():
        edge_loop(xl1_hbm)

    plsc.subcore_barrier()

    def out_body(cc, _):
        r0 = sid * (NP2 // 16) + cc * KE
        @pl.when(c == 0)
        def _o0():
            pltpu.sync_copy(accum_sh.at[pl.ds(r0, KE)], h0_hbm.at[pl.ds(r0, KE)])
        @pl.when(c == 1)
        def _o1():
            pltpu.sync_copy(accum_sh.at[pl.ds(r0, KE)], h1_hbm.at[pl.ds(r0, KE)])
        return _
    lax.fori_loop(0, NP2 // 16 // KE, out_body, None)

    @pl.when(c == 0)
    def _den():
        col0 = sid * (NP3 // 16)
        pltpu.sync_copy(denp_hbm.at[pl.ds(col0, NP3 // 16)], dsum)

        def r_body(r, _):
            pltpu.sync_copy(denp_hbm.at[pl.ds(r * NP3 + col0, NP3 // 16)], dbuf)

            def j_body(j, __):
                sl = pl.ds(j * 16, 16)
                dsum[sl] = dsum[sl] + dbuf[sl]
                return __
            lax.fori_loop(0, NP3 // 16 // 16, j_body, None, unroll=4)
            return _
        lax.fori_loop(1, NTILE, r_body, None)
        pltpu.sync_copy(dsum, den_hbm.at[pl.ds(col0, NP3 // 16)])


def _stage2(src, dst, xl0, xl1, xr, att):
    exw, denp = pl.kernel(
        _gat_pass1,
        out_type=[jax.ShapeDtypeStruct((EP,), jnp.float32),
                  jax.ShapeDtypeStruct((NTILE * NP3,), jnp.float32)],
        mesh=_sc_mesh(),
        compiler_params=pltpu.CompilerParams(needs_layout_passes=False, use_tc_tiling_on_sc=False),
        scratch_types=[
            pltpu.VMEM((KE,), jnp.int32), pltpu.VMEM((KE,), jnp.int32),
            pltpu.VMEM((KE, HL), jnp.float32), pltpu.VMEM((KE, HL), jnp.float32),
            pltpu.VMEM((KE, HID), jnp.float32), pltpu.VMEM((KE * 16,), jnp.float32),
            pltpu.VMEM((KE,), jnp.float32), pltpu.VMEM((HID,), jnp.float32),
            pltpu.VMEM((NP3,), jnp.float32),
            pltpu.SemaphoreType.DMA,
        ],
    )(src, dst, xl0, xl1, xr, att)

    h0, h1, den = pl.kernel(
        _gat_pass2,
        out_type=[jax.ShapeDtypeStruct((NP2, HL), jnp.float32),
                  jax.ShapeDtypeStruct((NP2, HL), jnp.float32),
                  jax.ShapeDtypeStruct((NP3,), jnp.float32)],
        mesh=_sc_mesh(),
        compiler_params=pltpu.CompilerParams(needs_layout_passes=False, use_tc_tiling_on_sc=False),
        scratch_types=[
            pltpu.VMEM((KE,), jnp.int32), pltpu.VMEM((KE,), jnp.int32),
            pltpu.VMEM((KE, HL), jnp.float32), pltpu.VMEM((KE, HL), jnp.float32),
            pltpu.VMEM((KE,), jnp.float32),
            pltpu.VMEM((NP3 // 16,), jnp.float32),
            pltpu.VMEM((NP3 // 16,), jnp.float32),
            pltpu.VMEM_SHARED((NP2, HL), jnp.float32),
            pltpu.SemaphoreType.DMA,
        ],
    )(src, dst, xl0, xl1, exw, denp)
    return h0, h1, den


BN3 = 2000


def _mlp_block(h0_ref, h1_ref, den_ref, bgat_ref, ggat_ref, begat_ref,
               w1_ref, b1_ref, g1_ref, be1_ref,
               w2_ref, b2_ref, g2_ref, be2_ref,
               w3_ref, b3_ref, out_ref,
               s1, q1, s2, q2, s3, q3):
    p = pl.program_id(0)
    i = pl.program_id(1)
    invn = jnp.float32(1.0 / N)

    @pl.when((p == 0) & (i == 0))
    def _init():
        s1[...] = jnp.zeros_like(s1)
        q1[...] = jnp.zeros_like(q1)
        s2[...] = jnp.zeros_like(s2)
        q2[...] = jnp.zeros_like(q2)
        s3[...] = jnp.zeros_like(s3)
        q3[...] = jnp.zeros_like(q3)

    def gat_out():
        h = jnp.concatenate([h0_ref[...], h1_ref[...]], axis=1)
        return h / (den_ref[...] + 1e-16) + bgat_ref[...]

    def bnrelu(z, s, q, g_ref, be_ref):
        m = s[...] * invn
        v = q[...] * invn - m * m
        return jax.nn.relu((z - m) / jnp.sqrt(v + 1e-5) * g_ref[...] + be_ref[...])

    @pl.when(p == 0)
    def _p0():
        h = gat_out()
        s1[...] += jnp.sum(h, axis=0, keepdims=True)
        q1[...] += jnp.sum(h * h, axis=0, keepdims=True)
        out_ref[...] = jnp.zeros_like(out_ref)

    @pl.when(p == 1)
    def _p1():
        h = bnrelu(gat_out(), s1, q1, ggat_ref, begat_ref)
        z = jnp.dot(h, w1_ref[...], preferred_element_type=jnp.float32) + b1_ref[...]
        s2[...] += jnp.sum(z, axis=0, keepdims=True)
        q2[...] += jnp.sum(z * z, axis=0, keepdims=True)

    @pl.when(p == 2)
    def _p2():
        h = bnrelu(gat_out(), s1, q1, ggat_ref, begat_ref)
        z = jnp.dot(h, w1_ref[...], preferred_element_type=jnp.float32) + b1_ref[...]
        h2 = bnrelu(z, s2, q2, g1_ref, be1_ref)
        z2 = jnp.dot(h2, w2_ref[...], preferred_element_type=jnp.float32) + b2_ref[...]
        s3[...] += jnp.sum(z2, axis=0, keepdims=True)
        q3[...] += jnp.sum(z2 * z2, axis=0, keepdims=True)

    @pl.when(p == 3)
    def _p3():
        h = bnrelu(gat_out(), s1, q1, ggat_ref, begat_ref)
        z = jnp.dot(h, w1_ref[...], preferred_element_type=jnp.float32) + b1_ref[...]
        h2 = bnrelu(z, s2, q2, g1_ref, be1_ref)
        z2 = jnp.dot(h2, w2_ref[...], preferred_element_type=jnp.float32) + b2_ref[...]
        h3 = bnrelu(z2, s3, q3, g2_ref, be2_ref)
        out_ref[...] = jnp.dot(h3, w3_ref[...], preferred_element_type=jnp.float32) + b3_ref[...]


def _stage3(h0, h1, den, bgat, ggat, begat, w1, b1, g1, be1,
            w2, b2, g2, be2, w3, b3):
    nblk = N // BN3
    full = lambda shape: pl.BlockSpec(shape, lambda p, i: (0,) * len(shape))
    blk = lambda shape: pl.BlockSpec(shape, lambda p, i: (i,) + (0,) * (len(shape) - 1))
    return pl.pallas_call(
        _mlp_block,
        grid=(4, nblk),
        in_specs=[
            blk((BN3, HL)), blk((BN3, HL)), blk((BN3, 1)),
            full((1, HID)), full((1, HID)), full((1, HID)),
            full((HID, 64)), full((1, 64)), full((1, 64)), full((1, 64)),
            full((64, 16)), full((1, 16)), full((1, 16)), full((1, 16)),
            full((16, 2)), full((1, 2)),
        ],
        out_specs=blk((BN3, 2)),
        out_shape=jax.ShapeDtypeStruct((N, 2), jnp.float32),
        scratch_shapes=[pltpu.VMEM((1, HID), jnp.float32),
                        pltpu.VMEM((1, HID), jnp.float32),
                        pltpu.VMEM((1, 64), jnp.float32),
                        pltpu.VMEM((1, 64), jnp.float32),
                        pltpu.VMEM((1, 16), jnp.float32),
                        pltpu.VMEM((1, 16), jnp.float32)],
    )(h0, h1, den, bgat, ggat, begat, w1, b1, g1, be1, w2, b2, g2, be2, w3, b3)


def kernel(x, edge_index, block_instructions, lengths, emb, W_ih_f, W_hh_f, b_ih_f, b_hh_f, W_ih_b, W_hh_b, b_ih_b, b_hh_b, W_l, b_l, W_r, b_r, att, bias_gat, g_gat, be_gat, W1, b1, g1, be1, W2, b2, g2, be2, W3, b3):
    tf = emb @ W_ih_f.T + (b_ih_f + b_hh_f)[None, :]
    tb = emb @ W_ih_b.T + (b_ih_b + b_hh_b)[None, :]
    whf = W_hh_f.T
    whb = W_hh_b.T
    idxm = jnp.clip(lengths[:, None] - 1 - jnp.arange(L)[None, :], 0, L - 1)
    rinst = jnp.take_along_axis(block_instructions, idxm, axis=1)
    lenf = lengths[:, None].astype(jnp.float32)
    wlx = W_l.T[:VOCAB, :]
    wlp = W_l.T[VOCAB:, :]
    wrx = W_r.T[:VOCAB, :]
    wrp = W_r.T[VOCAB:, :]

    xl0, xl1, xr = _stage1(block_instructions, rinst, lenf, x, tf, tb,
                           whf, whb, wlx, wlp, b_l[None, :], wrx, wrp,
                           b_r[None, :])

    loop = jnp.arange(N, dtype=jnp.int32)
    pad = jnp.zeros((EP - EACT,), jnp.int32)
    src = jnp.concatenate([edge_index[0], loop, pad])
    dst = jnp.concatenate([edge_index[1], loop, pad])

    h0, h1, den = _stage2(src, dst, xl0, xl1, xr, att)

    return _stage3(h0, h1, den[:, None], bias_gat[None, :], g_gat[None, :],
                   be_gat[None, :], W1.T, b1[None, :], g1[None, :],
                   be1[None, :], W2.T, b2[None, :], g2[None, :],
                   be2[None, :], W3.T, b3[None, :])


# speedup vs baseline: 10.7872x; 1.4488x over previous
import functools

import jax
import jax.numpy as jnp
from jax import lax
from jax.experimental import pallas as pl
from jax.experimental.pallas import tpu as pltpu
from jax.experimental.pallas import tpu_sc as plsc

N = 50000
E = 800000
VOCAB = 32
EMB = 128
L = 20
HL = 32
HID = 64

BN1 = 1000


def _lstm_block(inst_ref, rinst_ref, lenf_ref, x_ref,
                tcat_ref, wcat_ref,
                wlx_ref, wlp_ref, bl_ref, wrx_ref, wrp_ref, br_ref,
                xl0_ref, xl1_ref, xr_ref):
    bn = inst_ref.shape[0]
    lenf = lenf_ref[...]
    V2 = 2 * (VOCAB + 2)

    tcat = tcat_ref[...]
    wcat = wcat_ref[...]
    iot = lax.broadcasted_iota(jnp.int32, (bn, V2), 1)
    h = jnp.zeros((bn, 2 * HL), jnp.float32)
    c = jnp.zeros((bn, 2 * HL), jnp.float32)
    acc = jnp.zeros((bn, 2 * HL), jnp.float32)
    for t in range(L):
        colf = inst_ref[:, t:t + 1]
        colb = rinst_ref[:, t:t + 1] + (VOCAB + 2)
        oh = ((colf == iot) | (colb == iot)).astype(jnp.float32)
        gates = (jnp.dot(oh, tcat, preferred_element_type=jnp.float32)
                 + jnp.dot(h, wcat, preferred_element_type=jnp.float32))
        i = jax.nn.sigmoid(gates[:, 0:64])
        f = jax.nn.sigmoid(gates[:, 64:128])
        g = jnp.tanh(gates[:, 128:192])
        o = jax.nn.sigmoid(gates[:, 192:256])
        c = f * c + i * g
        h = o * jnp.tanh(c)
        mask = (jnp.float32(t) < lenf).astype(jnp.float32)
        acc = acc + h * mask

    pooled = acc * (1.0 / lenf)
    x = x_ref[...]
    xl = (jnp.dot(x, wlx_ref[...], preferred_element_type=jnp.float32)
          + jnp.dot(pooled, wlp_ref[...], preferred_element_type=jnp.float32)
          + bl_ref[...])
    xr = (jnp.dot(x, wrx_ref[...], preferred_element_type=jnp.float32)
          + jnp.dot(pooled, wrp_ref[...], preferred_element_type=jnp.float32)
          + br_ref[...])
    xl0_ref[...] = xl[:, 0:HL]
    xl1_ref[...] = xl[:, HL:HID]
    xr_ref[...] = xr


def _stage1(inst, rinst, lenf, x, tcat, wcat,
            wlx, wlp, bl, wrx, wrp, br):
    nblk = N // BN1
    full = lambda shape: pl.BlockSpec(shape, lambda i: (0,) * len(shape))
    blk = lambda shape: pl.BlockSpec(shape, lambda i: (i,) + (0,) * (len(shape) - 1))
    return pl.pallas_call(
        _lstm_block,
        grid=(nblk,),
        in_specs=[
            blk((BN1, L)), blk((BN1, L)), blk((BN1, 1)), blk((BN1, VOCAB)),
            full((2 * (VOCAB + 2), 256)), full((2 * HL, 256)),
            full((VOCAB, HID)), full((2 * HL, HID)), full((1, HID)),
            full((VOCAB, HID)), full((2 * HL, HID)), full((1, HID)),
        ],
        out_specs=[blk((BN1, HL)), blk((BN1, HL)), blk((BN1, HID))],
        out_shape=[
            jax.ShapeDtypeStruct((N, HL), jnp.float32),
            jax.ShapeDtypeStruct((N, HL), jnp.float32),
            jax.ShapeDtypeStruct((N, HID), jnp.float32),
        ],
    )(inst, rinst, lenf, x, tcat, wcat, wlx, wlp, bl, wrx, wrp, br)


def _paired_tables(tf, tb, whfT, whbT):
    z34 = jnp.zeros((VOCAB + 2, HL), jnp.float32)
    z32 = jnp.zeros((HL, HL), jnp.float32)
    tblocks, wblocks = [], []
    for gi in range(4):
        sl = slice(gi * HL, (gi + 1) * HL)
        tblocks.append(jnp.concatenate([
            jnp.concatenate([tf[:, sl], z34], axis=1),
            jnp.concatenate([z34, tb[:, sl]], axis=1)], axis=0))
        wblocks.append(jnp.concatenate([
            jnp.concatenate([whfT[:, sl], z32], axis=1),
            jnp.concatenate([z32, whbT[:, sl]], axis=1)], axis=0))
    return (jnp.concatenate(tblocks, axis=1),
            jnp.concatenate(wblocks, axis=1))


EACT = E + N
EP = 851968
KE = 128
NTILE = 32
NP2 = 51200
NP3 = 50176

_P1_CHUNKS = EP // NTILE // KE
_P2_CHUNKS = EP // 16 // KE
def _sc_mesh():
    return plsc.VectorSubcoreMesh(core_axis_name="c", subcore_axis_name="s",
                                  num_cores=2, num_subcores=16)


def _gat_pass1(src_hbm, dst_hbm, xl0_hbm, xl1_hbm, xr_hbm, att_hbm,
               exw_hbm, denp_hbm,
               sidx, didx, rl0, rl1, rr, sbuf, exbuf, attv, denloc, sem):
    wid = lax.axis_index("s") * 2 + lax.axis_index("c")
    pltpu.sync_copy(att_hbm, attv)
    iota = lax.iota(jnp.int32, 16)

    def zero_body(z, _):
        denloc[pl.ds(z * 16, 16)] = jnp.zeros((16,), jnp.float32)
        return _
    lax.fori_loop(0, NP3 // 16, zero_body, None)

    att_g = [attv[pl.ds(fg * 16, 16)] for fg in range(4)]

    def chunk_body(k, _):
        base = wid * (_P1_CHUNKS * KE) + k * KE
        pltpu.sync_copy(src_hbm.at[pl.ds(base, KE)], sidx)
        pltpu.sync_copy(dst_hbm.at[pl.ds(base, KE)], didx)
        c0 = pltpu.async_copy(xl0_hbm.at[sidx], rl0, sem)
        c1 = pltpu.async_copy(xl1_hbm.at[sidx], rl1, sem)
        c2 = pltpu.async_copy(xr_hbm.at[didx], rr, sem)
        c0.wait()
        c1.wait()
        c2.wait()

        def row_body(row, _):
            ps = jnp.zeros((16,), jnp.float32)
            for fg in range(4):
                if fg < 2:
                    a = rl0[row, pl.ds(fg * 16, 16)]
                else:
                    a = rl1[row, pl.ds((fg - 2) * 16, 16)]
                m = a + rr[row, pl.ds(fg * 16, 16)]
                m = jnp.maximum(m, m * jnp.float32(0.2))
                ps = ps + m * att_g[fg]
            sbuf[pl.ds(row * 16, 16)] = ps
            return _
        lax.fori_loop(0, KE, row_body, None, unroll=4)

        for g in range(8):
            flat = (iota + g * 16) * 16
            e = jnp.zeros((16,), jnp.float32)
            for l in range(16):
                e = e + plsc.load_gather(sbuf, [flat + l])
            ex = jnp.exp(e)
            exbuf[pl.ds(g * 16, 16)] = ex
            dv = didx[pl.ds(g * 16, 16)]
            valid = (base + g * 16 + iota) < EACT
            dv = jnp.where(valid, dv, jnp.int32(NP3 - 1))
            plsc.addupdate_scatter(denloc, [dv], ex)
        pltpu.sync_copy(exbuf, exw_hbm.at[pl.ds(base, KE)])
        return _

    lax.fori_loop(0, _P1_CHUNKS, chunk_body, None)
    pltpu.sync_copy(denloc, denp_hbm.at[pl.ds(wid * NP3, NP3)])


def _gat_pass2(src_hbm, dst_hbm, xl0_hbm, xl1_hbm, exw_hbm, denp_hbm,
               h0_hbm, h1_hbm, den_hbm,
               sidx, d2, rows, wrows, exbuf, dbuf, dsum, accum_sh, sem):
    c = lax.axis_index("c")
    sid = lax.axis_index("s")
    iota = lax.iota(jnp.int32, 16)

    def zw(row, _):
        wrows[row, pl.ds(0, 16)] = jnp.zeros((16,), jnp.float32)
        wrows[row, pl.ds(16, 16)] = jnp.zeros((16,), jnp.float32)
        return _
    lax.fori_loop(0, KE, zw, None, unroll=8)

    def zacc(cc, _):
        pltpu.sync_copy(wrows, accum_sh.at[pl.ds(sid * (NP2 // 16) + cc * KE, KE)])
        return _
    lax.fori_loop(0, NP2 // 16 // KE, zacc, None)
    plsc.subcore_barrier()

    def edge_loop(xlh_hbm):
        def chunk_body(k, _):
            base = sid * (_P2_CHUNKS * KE) + k * KE
            pltpu.sync_copy(src_hbm.at[pl.ds(base, KE)], sidx)
            pltpu.sync_copy(exw_hbm.at[pl.ds(base, KE)], exbuf)
            cg = pltpu.async_copy(xlh_hbm.at[sidx], rows, sem)
            pltpu.sync_copy(dst_hbm.at[pl.ds(base, KE)], d2)
            for g in range(8):
                dv = d2[pl.ds(g * 16, 16)]
                valid = (base + g * 16 + iota) < EACT
                d2[pl.ds(g * 16, 16)] = jnp.where(valid, dv, jnp.int32(NP2 - 1))
            cg.wait()

            def grp_body(g, _):
                exv = exbuf[pl.ds(g * 16, 16)]
                for l in range(16):
                    row = g * 16 + l
                    ex = exv[l]
                    wrows[row, pl.ds(0, 16)] = rows[row, pl.ds(0, 16)] * ex
                    wrows[row, pl.ds(16, 16)] = rows[row, pl.ds(16, 16)] * ex
                return _
            lax.fori_loop(0, KE // 16, grp_body, None)
            pltpu.sync_copy(wrows, accum_sh.at[d2], add=True)
            return _
        lax.fori_loop(0, _P2_CHUNKS, chunk_body, None)

    @pl.when(c == 0)
    def _sc0():
        edge_loop(xl0_hbm)

    @pl.when(c == 1)
    def _sc1():
        edge_loop(xl1_hbm)

    plsc.subcore_barrier()

    def out_body(cc, _):
        r0 = sid * (NP2 // 16) + cc * KE
        @pl.when(c == 0)
        def _o0():
            pltpu.sync_copy(accum_sh.at[pl.ds(r0, KE)], h0_hbm.at[pl.ds(r0, KE)])
        @pl.when(c == 1)
        def _o1():
            pltpu.sync_copy(accum_sh.at[pl.ds(r0, KE)], h1_hbm.at[pl.ds(r0, KE)])
        return _
    lax.fori_loop(0, NP2 // 16 // KE, out_body, None)

    @pl.when(c == 0)
    def _den():
        col0 = sid * (NP3 // 16)
        pltpu.sync_copy(denp_hbm.at[pl.ds(col0, NP3 // 16)], dsum)

        def r_body(r, _):
            pltpu.sync_copy(denp_hbm.at[pl.ds(r * NP3 + col0, NP3 // 16)], dbuf)

            def j_body(j, __):
                sl = pl.ds(j * 16, 16)
                dsum[sl] = dsum[sl] + dbuf[sl]
                return __
            lax.fori_loop(0, NP3 // 16 // 16, j_body, None, unroll=4)
            return _
        lax.fori_loop(1, NTILE, r_body, None)
        pltpu.sync_copy(dsum, den_hbm.at[pl.ds(col0, NP3 // 16)])


def _stage2(src, dst, xl0, xl1, xr, att):
    exw, denp = pl.kernel(
        _gat_pass1,
        out_type=[jax.ShapeDtypeStruct((EP,), jnp.float32),
                  jax.ShapeDtypeStruct((NTILE * NP3,), jnp.float32)],
        mesh=_sc_mesh(),
        compiler_params=pltpu.CompilerParams(needs_layout_passes=False, use_tc_tiling_on_sc=False),
        scratch_types=[
            pltpu.VMEM((KE,), jnp.int32), pltpu.VMEM((KE,), jnp.int32),
            pltpu.VMEM((KE, HL), jnp.float32), pltpu.VMEM((KE, HL), jnp.float32),
            pltpu.VMEM((KE, HID), jnp.float32), pltpu.VMEM((KE * 16,), jnp.float32),
            pltpu.VMEM((KE,), jnp.float32), pltpu.VMEM((HID,), jnp.float32),
            pltpu.VMEM((NP3,), jnp.float32),
            pltpu.SemaphoreType.DMA,
        ],
    )(src, dst, xl0, xl1, xr, att)

    h0, h1, den = pl.kernel(
        _gat_pass2,
        out_type=[jax.ShapeDtypeStruct((NP2, HL), jnp.float32),
                  jax.ShapeDtypeStruct((NP2, HL), jnp.float32),
                  jax.ShapeDtypeStruct((NP3,), jnp.float32)],
        mesh=_sc_mesh(),
        compiler_params=pltpu.CompilerParams(needs_layout_passes=False, use_tc_tiling_on_sc=False),
        scratch_types=[
            pltpu.VMEM((KE,), jnp.int32), pltpu.VMEM((KE,), jnp.int32),
            pltpu.VMEM((KE, HL), jnp.float32), pltpu.VMEM((KE, HL), jnp.float32),
            pltpu.VMEM((KE,), jnp.float32),
            pltpu.VMEM((NP3 // 16,), jnp.float32),
            pltpu.VMEM((NP3 // 16,), jnp.float32),
            pltpu.VMEM_SHARED((NP2, HL), jnp.float32),
            pltpu.SemaphoreType.DMA,
        ],
    )(src, dst, xl0, xl1, exw, denp)
    return h0, h1, den


BN3 = 2000


def _mlp_block(h0_ref, h1_ref, den_ref, bgat_ref, ggat_ref, begat_ref,
               w1_ref, b1_ref, g1_ref, be1_ref,
               w2_ref, b2_ref, g2_ref, be2_ref,
               w3_ref, b3_ref, out_ref,
               s1, q1, s2, q2, s3, q3):
    p = pl.program_id(0)
    i = pl.program_id(1)
    invn = jnp.float32(1.0 / N)

    @pl.when((p == 0) & (i == 0))
    def _init():
        s1[...] = jnp.zeros_like(s1)
        q1[...] = jnp.zeros_like(q1)
        s2[...] = jnp.zeros_like(s2)
        q2[...] = jnp.zeros_like(q2)
        s3[...] = jnp.zeros_like(s3)
        q3[...] = jnp.zeros_like(q3)

    def gat_out():
        h = jnp.concatenate([h0_ref[...], h1_ref[...]], axis=1)
        return h / (den_ref[...] + 1e-16) + bgat_ref[...]

    def bnrelu(z, s, q, g_ref, be_ref):
        m = s[...] * invn
        v = q[...] * invn - m * m
        return jax.nn.relu((z - m) / jnp.sqrt(v + 1e-5) * g_ref[...] + be_ref[...])

    @pl.when(p == 0)
    def _p0():
        h = gat_out()
        s1[...] += jnp.sum(h, axis=0, keepdims=True)
        q1[...] += jnp.sum(h * h, axis=0, keepdims=True)
        out_ref[...] = jnp.zeros_like(out_ref)

    @pl.when(p == 1)
    def _p1():
        h = bnrelu(gat_out(), s1, q1, ggat_ref, begat_ref)
        z = jnp.dot(h, w1_ref[...], preferred_element_type=jnp.float32) + b1_ref[...]
        s2[...] += jnp.sum(z, axis=0, keepdims=True)
        q2[...] += jnp.sum(z * z, axis=0, keepdims=True)

    @pl.when(p == 2)
    def _p2():
        h = bnrelu(gat_out(), s1, q1, ggat_ref, begat_ref)
        z = jnp.dot(h, w1_ref[...], preferred_element_type=jnp.float32) + b1_ref[...]
        h2 = bnrelu(z, s2, q2, g1_ref, be1_ref)
        z2 = jnp.dot(h2, w2_ref[...], preferred_element_type=jnp.float32) + b2_ref[...]
        s3[...] += jnp.sum(z2, axis=0, keepdims=True)
        q3[...] += jnp.sum(z2 * z2, axis=0, keepdims=True)

    @pl.when(p == 3)
    def _p3():
        h = bnrelu(gat_out(), s1, q1, ggat_ref, begat_ref)
        z = jnp.dot(h, w1_ref[...], preferred_element_type=jnp.float32) + b1_ref[...]
        h2 = bnrelu(z, s2, q2, g1_ref, be1_ref)
        z2 = jnp.dot(h2, w2_ref[...], preferred_element_type=jnp.float32) + b2_ref[...]
        h3 = bnrelu(z2, s3, q3, g2_ref, be2_ref)
        out_ref[...] = jnp.dot(h3, w3_ref[...], preferred_element_type=jnp.float32) + b3_ref[...]


def _stage3(h0, h1, den, bgat, ggat, begat, w1, b1, g1, be1,
            w2, b2, g2, be2, w3, b3):
    nblk = N // BN3
    full = lambda shape: pl.BlockSpec(shape, lambda p, i: (0,) * len(shape))
    blk = lambda shape: pl.BlockSpec(shape, lambda p, i: (i,) + (0,) * (len(shape) - 1))
    return pl.pallas_call(
        _mlp_block,
        grid=(4, nblk),
        in_specs=[
            blk((BN3, HL)), blk((BN3, HL)), blk((BN3, 1)),
            full((1, HID)), full((1, HID)), full((1, HID)),
            full((HID, 64)), full((1, 64)), full((1, 64)), full((1, 64)),
            full((64, 16)), full((1, 16)), full((1, 16)), full((1, 16)),
            full((16, 2)), full((1, 2)),
        ],
        out_specs=blk((BN3, 2)),
        out_shape=jax.ShapeDtypeStruct((N, 2), jnp.float32),
        scratch_shapes=[pltpu.VMEM((1, HID), jnp.float32),
                        pltpu.VMEM((1, HID), jnp.float32),
                        pltpu.VMEM((1, 64), jnp.float32),
                        pltpu.VMEM((1, 64), jnp.float32),
                        pltpu.VMEM((1, 16), jnp.float32),
                        pltpu.VMEM((1, 16), jnp.float32)],
    )(h0, h1, den, bgat, ggat, begat, w1, b1, g1, be1, w2, b2, g2, be2, w3, b3)


def kernel(x, edge_index, block_instructions, lengths, emb, W_ih_f, W_hh_f, b_ih_f, b_hh_f, W_ih_b, W_hh_b, b_ih_b, b_hh_b, W_l, b_l, W_r, b_r, att, bias_gat, g_gat, be_gat, W1, b1, g1, be1, W2, b2, g2, be2, W3, b3):
    tf = emb @ W_ih_f.T + (b_ih_f + b_hh_f)[None, :]
    tb = emb @ W_ih_b.T + (b_ih_b + b_hh_b)[None, :]
    tcat, wcat = _paired_tables(tf, tb, W_hh_f.T, W_hh_b.T)
    idxm = jnp.clip(lengths[:, None] - 1 - jnp.arange(L)[None, :], 0, L - 1)
    rinst = jnp.take_along_axis(block_instructions, idxm, axis=1)
    lenf = lengths[:, None].astype(jnp.float32)
    wlx = W_l.T[:VOCAB, :]
    wlp = W_l.T[VOCAB:, :]
    wrx = W_r.T[:VOCAB, :]
    wrp = W_r.T[VOCAB:, :]

    xl0, xl1, xr = _stage1(block_instructions, rinst, lenf, x, tcat, wcat,
                           wlx, wlp, b_l[None, :], wrx, wrp, b_r[None, :])

    loop = jnp.arange(N, dtype=jnp.int32)
    pad = jnp.zeros((EP - EACT,), jnp.int32)
    src = jnp.concatenate([edge_index[0], loop, pad])
    dst = jnp.concatenate([edge_index[1], loop, pad])

    h0, h1, den = _stage2(src, dst, xl0, xl1, xr, att)

    return _stage3(h0, h1, den[:, None], bias_gat[None, :], g_gat[None, :],
                   be_gat[None, :], W1.T, b1[None, :], g1[None, :],
                   be1[None, :], W2.T, b2[None, :], g2[None, :],
                   be2[None, :], W3.T, b3[None, :])


# speedup vs baseline: 11.8877x; 1.1020x over previous
import functools

import jax
import jax.numpy as jnp
from jax import lax
from jax.experimental import pallas as pl
from jax.experimental.pallas import tpu as pltpu
from jax.experimental.pallas import tpu_sc as plsc

N = 50000
E = 800000
VOCAB = 32
EMB = 128
L = 20
HL = 32
HID = 64

BN1 = 1000


def _lstm_block(inst_ref, rinst_ref, lenf_ref, x_ref,
                tcat_ref, wcat_ref,
                wlx_ref, wlp_ref, bl_ref, wrx_ref, wrp_ref, br_ref,
                xl0_ref, xl1_ref, xr_ref):
    bn = inst_ref.shape[0]
    lenf = lenf_ref[...]
    V2 = 2 * (VOCAB + 2)

    tcat = tcat_ref[...]
    wcat = wcat_ref[...]
    iot = lax.broadcasted_iota(jnp.int32, (bn, V2), 1)
    h = jnp.zeros((bn, 2 * HL), jnp.float32)
    c = jnp.zeros((bn, 2 * HL), jnp.float32)
    acc = jnp.zeros((bn, 2 * HL), jnp.float32)
    for t in range(L):
        colf = inst_ref[:, t:t + 1]
        colb = rinst_ref[:, t:t + 1] + (VOCAB + 2)
        oh = ((colf == iot) | (colb == iot)).astype(jnp.float32)
        gates = (jnp.dot(oh, tcat, preferred_element_type=jnp.float32)
                 + jnp.dot(h, wcat, preferred_element_type=jnp.float32))
        i = jax.nn.sigmoid(gates[:, 0:64])
        f = jax.nn.sigmoid(gates[:, 64:128])
        g = jnp.tanh(gates[:, 128:192])
        o = jax.nn.sigmoid(gates[:, 192:256])
        c = f * c + i * g
        h = o * jnp.tanh(c)
        mask = (jnp.float32(t) < lenf).astype(jnp.float32)
        acc = acc + h * mask

    pooled = acc * (1.0 / lenf)
    x = x_ref[...]
    xl = (jnp.dot(x, wlx_ref[...], preferred_element_type=jnp.float32)
          + jnp.dot(pooled, wlp_ref[...], preferred_element_type=jnp.float32)
          + bl_ref[...])
    xr = (jnp.dot(x, wrx_ref[...], preferred_element_type=jnp.float32)
          + jnp.dot(pooled, wrp_ref[...], preferred_element_type=jnp.float32)
          + br_ref[...])
    xl0_ref[...] = xl[:, 0:HL]
    xl1_ref[...] = xl[:, HL:HID]
    xr_ref[...] = xr


def _stage1(inst, rinst, lenf, x, tcat, wcat,
            wlx, wlp, bl, wrx, wrp, br):
    nblk = N // BN1
    full = lambda shape: pl.BlockSpec(shape, lambda i: (0,) * len(shape))
    blk = lambda shape: pl.BlockSpec(shape, lambda i: (i,) + (0,) * (len(shape) - 1))
    return pl.pallas_call(
        _lstm_block,
        grid=(nblk,),
        in_specs=[
            blk((BN1, L)), blk((BN1, L)), blk((BN1, 1)), blk((BN1, VOCAB)),
            full((2 * (VOCAB + 2), 256)), full((2 * HL, 256)),
            full((VOCAB, HID)), full((2 * HL, HID)), full((1, HID)),
            full((VOCAB, HID)), full((2 * HL, HID)), full((1, HID)),
        ],
        out_specs=[blk((BN1, HL)), blk((BN1, HL)), blk((BN1, HID))],
        out_shape=[
            jax.ShapeDtypeStruct((N, HL), jnp.float32),
            jax.ShapeDtypeStruct((N, HL), jnp.float32),
            jax.ShapeDtypeStruct((N, HID), jnp.float32),
        ],
    )(inst, rinst, lenf, x, tcat, wcat, wlx, wlp, bl, wrx, wrp, br)


def _paired_tables(tf, tb, whfT, whbT):
    z34 = jnp.zeros((VOCAB + 2, HL), jnp.float32)
    z32 = jnp.zeros((HL, HL), jnp.float32)
    tblocks, wblocks = [], []
    for gi in range(4):
        sl = slice(gi * HL, (gi + 1) * HL)
        tblocks.append(jnp.concatenate([
            jnp.concatenate([tf[:, sl], z34], axis=1),
            jnp.concatenate([z34, tb[:, sl]], axis=1)], axis=0))
        wblocks.append(jnp.concatenate([
            jnp.concatenate([whfT[:, sl], z32], axis=1),
            jnp.concatenate([z32, whbT[:, sl]], axis=1)], axis=0))
    return (jnp.concatenate(tblocks, axis=1),
            jnp.concatenate(wblocks, axis=1))


EACT = E + N
EP = 851968
KE = 128
NTILE = 32
NP2 = 51200
NP3 = 50176

_P1_CHUNKS = EP // NTILE // KE
_P2_CHUNKS = EP // 16 // KE
def _sc_mesh():
    return plsc.VectorSubcoreMesh(core_axis_name="c", subcore_axis_name="s",
                                  num_cores=2, num_subcores=16)


def _gat_pass1(src_hbm, dst_hbm, xl0_hbm, xl1_hbm, xr_hbm, att_hbm,
               exw_hbm, denp_hbm,
               sidxA, didxA, rl0A, rl1A, rrA,
               sidxB, didxB, rl0B, rl1B, rrB,
               sbuf, exbuf, attv, denloc, semA, semB):
    wid = lax.axis_index("s") * 2 + lax.axis_index("c")
    pltpu.sync_copy(att_hbm, attv)
    iota = lax.iota(jnp.int32, 16)
    bufs = [(sidxA, didxA, rl0A, rl1A, rrA, semA),
            (sidxB, didxB, rl0B, rl1B, rrB, semB)]

    def zero_body(z, _):
        denloc[pl.ds(z * 16, 16)] = jnp.zeros((16,), jnp.float32)
        return _
    lax.fori_loop(0, NP3 // 16, zero_body, None)

    att_g = [attv[pl.ds(fg * 16, 16)] for fg in range(4)]

    def start(cidx, buf):
        sidx, didx, rl0, rl1, rr, sem = buf
        base = wid * (_P1_CHUNKS * KE) + cidx * KE
        pltpu.sync_copy(src_hbm.at[pl.ds(base, KE)], sidx)
        pltpu.sync_copy(dst_hbm.at[pl.ds(base, KE)], didx)
        pltpu.async_copy(xl0_hbm.at[sidx], rl0, sem)
        pltpu.async_copy(xl1_hbm.at[sidx], rl1, sem)
        pltpu.async_copy(xr_hbm.at[didx], rr, sem)

    def drain(buf):
        sidx, didx, rl0, rl1, rr, sem = buf
        pltpu.make_async_copy(xl0_hbm.at[pl.ds(0, KE)], rl0, sem).wait()
        pltpu.make_async_copy(xl1_hbm.at[pl.ds(0, KE)], rl1, sem).wait()
        pltpu.make_async_copy(xr_hbm.at[pl.ds(0, KE)], rr, sem).wait()

    def compute(cidx, buf):
        sidx, didx, rl0, rl1, rr, sem = buf
        base = wid * (_P1_CHUNKS * KE) + cidx * KE

        def row_body(row, _):
            ps = jnp.zeros((16,), jnp.float32)
            for fg in range(4):
                if fg < 2:
                    a = rl0[row, pl.ds(fg * 16, 16)]
                else:
                    a = rl1[row, pl.ds((fg - 2) * 16, 16)]
                m = a + rr[row, pl.ds(fg * 16, 16)]
                m = jnp.maximum(m, m * jnp.float32(0.2))
                ps = ps + m * att_g[fg]
            sbuf[pl.ds(row * 16, 16)] = ps
            return _
        lax.fori_loop(0, KE, row_body, None, unroll=4)

        for g in range(8):
            flat = (iota + g * 16) * 16
            e = jnp.zeros((16,), jnp.float32)
            for l in range(16):
                e = e + plsc.load_gather(sbuf, [flat + l])
            ex = jnp.exp(e)
            exbuf[pl.ds(g * 16, 16)] = ex
            dv = didx[pl.ds(g * 16, 16)]
            valid = (base + g * 16 + iota) < EACT
            dv = jnp.where(valid, dv, jnp.int32(NP3 - 1))
            plsc.addupdate_scatter(denloc, [dv], ex)
        pltpu.sync_copy(exbuf, exw_hbm.at[pl.ds(base, KE)])

    start(0, bufs[0])

    def pair_body(k2, _):
        for b in range(2):
            cidx = k2 * 2 + b
            drain(bufs[b])
            start(jnp.minimum(cidx + 1, _P1_CHUNKS - 1), bufs[1 - b])
            compute(cidx, bufs[b])
        return _
    lax.fori_loop(0, _P1_CHUNKS // 2, pair_body, None)
    drain(bufs[0])
    pltpu.sync_copy(denloc, denp_hbm.at[pl.ds(wid * NP3, NP3)])


def _gat_pass2(src_hbm, dst_hbm, xl0_hbm, xl1_hbm, exw_hbm, denp_hbm,
               h0_hbm, h1_hbm, den_hbm,
               sidxA, rowsA, exbufA, sidxB, rowsB, exbufB,
               d2, wrows, dbuf, dsum, accum_sh, semA, semB):
    c = lax.axis_index("c")
    sid = lax.axis_index("s")
    iota = lax.iota(jnp.int32, 16)
    bufs = [(sidxA, rowsA, exbufA, semA), (sidxB, rowsB, exbufB, semB)]

    def zw(row, _):
        wrows[row, pl.ds(0, 16)] = jnp.zeros((16,), jnp.float32)
        wrows[row, pl.ds(16, 16)] = jnp.zeros((16,), jnp.float32)
        return _
    lax.fori_loop(0, KE, zw, None, unroll=8)

    def zacc(cc, _):
        pltpu.sync_copy(wrows, accum_sh.at[pl.ds(sid * (NP2 // 16) + cc * KE, KE)])
        return _
    lax.fori_loop(0, NP2 // 16 // KE, zacc, None)
    plsc.subcore_barrier()

    def edge_loop(xlh_hbm):
        def start(cidx, buf):
            sidx, rows, exbuf, sem = buf
            base = sid * (_P2_CHUNKS * KE) + cidx * KE
            pltpu.sync_copy(src_hbm.at[pl.ds(base, KE)], sidx)
            pltpu.sync_copy(exw_hbm.at[pl.ds(base, KE)], exbuf)
            pltpu.async_copy(xlh_hbm.at[sidx], rows, sem)

        def drain(buf):
            sidx, rows, exbuf, sem = buf
            pltpu.make_async_copy(xlh_hbm.at[pl.ds(0, KE)], rows, sem).wait()

        def compute(cidx, buf):
            sidx, rows, exbuf, sem = buf
            base = sid * (_P2_CHUNKS * KE) + cidx * KE
            pltpu.sync_copy(dst_hbm.at[pl.ds(base, KE)], d2)
            for g in range(8):
                dv = d2[pl.ds(g * 16, 16)]
                valid = (base + g * 16 + iota) < EACT
                d2[pl.ds(g * 16, 16)] = jnp.where(valid, dv, jnp.int32(NP2 - 1))

            def grp_body(g, _):
                exv = exbuf[pl.ds(g * 16, 16)]
                for l in range(16):
                    row = g * 16 + l
                    ex = exv[l]
                    wrows[row, pl.ds(0, 16)] = rows[row, pl.ds(0, 16)] * ex
                    wrows[row, pl.ds(16, 16)] = rows[row, pl.ds(16, 16)] * ex
                return _
            lax.fori_loop(0, KE // 16, grp_body, None)
            pltpu.sync_copy(wrows, accum_sh.at[d2], add=True)

        start(0, bufs[0])

        def pair_body(k2, _):
            for b in range(2):
                cidx = k2 * 2 + b
                drain(bufs[b])
                start(jnp.minimum(cidx + 1, _P2_CHUNKS - 1), bufs[1 - b])
                compute(cidx, bufs[b])
            return _
        lax.fori_loop(0, _P2_CHUNKS // 2, pair_body, None)
        drain(bufs[0])

    @pl.when(c == 0)
    def _sc0():
        edge_loop(xl0_hbm)

    @pl.when(c == 1)
    def _sc1():
        edge_loop(xl1_hbm)

    plsc.subcore_barrier()

    def out_body(cc, _):
        r0 = sid * (NP2 // 16) + cc * KE
        @pl.when(c == 0)
        def _o0():
            pltpu.sync_copy(accum_sh.at[pl.ds(r0, KE)], h0_hbm.at[pl.ds(r0, KE)])
        @pl.when(c == 1)
        def _o1():
            pltpu.sync_copy(accum_sh.at[pl.ds(r0, KE)], h1_hbm.at[pl.ds(r0, KE)])
        return _
    lax.fori_loop(0, NP2 // 16 // KE, out_body, None)

    @pl.when(c == 0)
    def _den():
        col0 = sid * (NP3 // 16)
        pltpu.sync_copy(denp_hbm.at[pl.ds(col0, NP3 // 16)], dsum)

        def r_body(r, _):
            pltpu.sync_copy(denp_hbm.at[pl.ds(r * NP3 + col0, NP3 // 16)], dbuf)

            def j_body(j, __):
                sl = pl.ds(j * 16, 16)
                dsum[sl] = dsum[sl] + dbuf[sl]
                return __
            lax.fori_loop(0, NP3 // 16 // 16, j_body, None, unroll=4)
            return _
        lax.fori_loop(1, NTILE, r_body, None)
        pltpu.sync_copy(dsum, den_hbm.at[pl.ds(col0, NP3 // 16)])


def _stage2(src, dst, xl0, xl1, xr, att):
    exw, denp = pl.kernel(
        _gat_pass1,
        out_type=[jax.ShapeDtypeStruct((EP,), jnp.float32),
                  jax.ShapeDtypeStruct((NTILE * NP3,), jnp.float32)],
        mesh=_sc_mesh(),
        compiler_params=pltpu.CompilerParams(needs_layout_passes=False, use_tc_tiling_on_sc=False),
        scratch_types=[
            pltpu.VMEM((KE,), jnp.int32), pltpu.VMEM((KE,), jnp.int32),
            pltpu.VMEM((KE, HL), jnp.float32), pltpu.VMEM((KE, HL), jnp.float32),
            pltpu.VMEM((KE, HID), jnp.float32),
            pltpu.VMEM((KE,), jnp.int32), pltpu.VMEM((KE,), jnp.int32),
            pltpu.VMEM((KE, HL), jnp.float32), pltpu.VMEM((KE, HL), jnp.float32),
            pltpu.VMEM((KE, HID), jnp.float32),
            pltpu.VMEM((KE * 16,), jnp.float32),
            pltpu.VMEM((KE,), jnp.float32), pltpu.VMEM((HID,), jnp.float32),
            pltpu.VMEM((NP3,), jnp.float32),
            pltpu.SemaphoreType.DMA, pltpu.SemaphoreType.DMA,
        ],
    )(src, dst, xl0, xl1, xr, att)

    h0, h1, den = pl.kernel(
        _gat_pass2,
        out_type=[jax.ShapeDtypeStruct((NP2, HL), jnp.float32),
                  jax.ShapeDtypeStruct((NP2, HL), jnp.float32),
                  jax.ShapeDtypeStruct((NP3,), jnp.float32)],
        mesh=_sc_mesh(),
        compiler_params=pltpu.CompilerParams(needs_layout_passes=False, use_tc_tiling_on_sc=False),
        scratch_types=[
            pltpu.VMEM((KE,), jnp.int32), pltpu.VMEM((KE, HL), jnp.float32),
            pltpu.VMEM((KE,), jnp.float32),
            pltpu.VMEM((KE,), jnp.int32), pltpu.VMEM((KE, HL), jnp.float32),
            pltpu.VMEM((KE,), jnp.float32),
            pltpu.VMEM((KE,), jnp.int32), pltpu.VMEM((KE, HL), jnp.float32),
            pltpu.VMEM((NP3 // 16,), jnp.float32),
            pltpu.VMEM((NP3 // 16,), jnp.float32),
            pltpu.VMEM_SHARED((NP2, HL), jnp.float32),
            pltpu.SemaphoreType.DMA, pltpu.SemaphoreType.DMA,
        ],
    )(src, dst, xl0, xl1, exw, denp)
    return h0, h1, den


BN3 = 2000


def _mlp_block(h0_ref, h1_ref, den_ref, bgat_ref, ggat_ref, begat_ref,
               w1_ref, b1_ref, g1_ref, be1_ref,
               w2_ref, b2_ref, g2_ref, be2_ref,
               w3_ref, b3_ref, out_ref,
               s1, q1, s2, q2, s3, q3):
    p = pl.program_id(0)
    i = pl.program_id(1)
    invn = jnp.float32(1.0 / N)

    @pl.when((p == 0) & (i == 0))
    def _init():
        s1[...] = jnp.zeros_like(s1)
        q1[...] = jnp.zeros_like(q1)
        s2[...] = jnp.zeros_like(s2)
        q2[...] = jnp.zeros_like(q2)
        s3[...] = jnp.zeros_like(s3)
        q3[...] = jnp.zeros_like(q3)

    def gat_out():
        h = jnp.concatenate([h0_ref[...], h1_ref[...]], axis=1)
        return h / (den_ref[...] + 1e-16) + bgat_ref[...]

    def bnrelu(z, s, q, g_ref, be_ref):
        m = s[...] * invn
        v = q[...] * invn - m * m
        return jax.nn.relu((z - m) / jnp.sqrt(v + 1e-5) * g_ref[...] + be_ref[...])

    @pl.when(p == 0)
    def _p0():
        h = gat_out()
        s1[...] += jnp.sum(h, axis=0, keepdims=True)
        q1[...] += jnp.sum(h * h, axis=0, keepdims=True)
        out_ref[...] = jnp.zeros_like(out_ref)

    @pl.when(p == 1)
    def _p1():
        h = bnrelu(gat_out(), s1, q1, ggat_ref, begat_ref)
        z = jnp.dot(h, w1_ref[...], preferred_element_type=jnp.float32) + b1_ref[...]
        s2[...] += jnp.sum(z, axis=0, keepdims=True)
        q2[...] += jnp.sum(z * z, axis=0, keepdims=True)

    @pl.when(p == 2)
    def _p2():
        h = bnrelu(gat_out(), s1, q1, ggat_ref, begat_ref)
        z = jnp.dot(h, w1_ref[...], preferred_element_type=jnp.float32) + b1_ref[...]
        h2 = bnrelu(z, s2, q2, g1_ref, be1_ref)
        z2 = jnp.dot(h2, w2_ref[...], preferred_element_type=jnp.float32) + b2_ref[...]
        s3[...] += jnp.sum(z2, axis=0, keepdims=True)
        q3[...] += jnp.sum(z2 * z2, axis=0, keepdims=True)

    @pl.when(p == 3)
    def _p3():
        h = bnrelu(gat_out(), s1, q1, ggat_ref, begat_ref)
        z = jnp.dot(h, w1_ref[...], preferred_element_type=jnp.float32) + b1_ref[...]
        h2 = bnrelu(z, s2, q2, g1_ref, be1_ref)
        z2 = jnp.dot(h2, w2_ref[...], preferred_element_type=jnp.float32) + b2_ref[...]
        h3 = bnrelu(z2, s3, q3, g2_ref, be2_ref)
        out_ref[...] = jnp.dot(h3, w3_ref[...], preferred_element_type=jnp.float32) + b3_ref[...]


def _stage3(h0, h1, den, bgat, ggat, begat, w1, b1, g1, be1,
            w2, b2, g2, be2, w3, b3):
    nblk = N // BN3
    full = lambda shape: pl.BlockSpec(shape, lambda p, i: (0,) * len(shape))
    blk = lambda shape: pl.BlockSpec(shape, lambda p, i: (i,) + (0,) * (len(shape) - 1))
    return pl.pallas_call(
        _mlp_block,
        grid=(4, nblk),
        in_specs=[
            blk((BN3, HL)), blk((BN3, HL)), blk((BN3, 1)),
            full((1, HID)), full((1, HID)), full((1, HID)),
            full((HID, 64)), full((1, 64)), full((1, 64)), full((1, 64)),
            full((64, 16)), full((1, 16)), full((1, 16)), full((1, 16)),
            full((16, 2)), full((1, 2)),
        ],
        out_specs=blk((BN3, 2)),
        out_shape=jax.ShapeDtypeStruct((N, 2), jnp.float32),
        scratch_shapes=[pltpu.VMEM((1, HID), jnp.float32),
                        pltpu.VMEM((1, HID), jnp.float32),
                        pltpu.VMEM((1, 64), jnp.float32),
                        pltpu.VMEM((1, 64), jnp.float32),
                        pltpu.VMEM((1, 16), jnp.float32),
                        pltpu.VMEM((1, 16), jnp.float32)],
    )(h0, h1, den, bgat, ggat, begat, w1, b1, g1, be1, w2, b2, g2, be2, w3, b3)


def kernel(x, edge_index, block_instructions, lengths, emb, W_ih_f, W_hh_f, b_ih_f, b_hh_f, W_ih_b, W_hh_b, b_ih_b, b_hh_b, W_l, b_l, W_r, b_r, att, bias_gat, g_gat, be_gat, W1, b1, g1, be1, W2, b2, g2, be2, W3, b3):
    tf = emb @ W_ih_f.T + (b_ih_f + b_hh_f)[None, :]
    tb = emb @ W_ih_b.T + (b_ih_b + b_hh_b)[None, :]
    tcat, wcat = _paired_tables(tf, tb, W_hh_f.T, W_hh_b.T)
    idxm = jnp.clip(lengths[:, None] - 1 - jnp.arange(L)[None, :], 0, L - 1)
    rinst = jnp.take_along_axis(block_instructions, idxm, axis=1)
    lenf = lengths[:, None].astype(jnp.float32)
    wlx = W_l.T[:VOCAB, :]
    wlp = W_l.T[VOCAB:, :]
    wrx = W_r.T[:VOCAB, :]
    wrp = W_r.T[VOCAB:, :]

    xl0, xl1, xr = _stage1(block_instructions, rinst, lenf, x, tcat, wcat,
                           wlx, wlp, b_l[None, :], wrx, wrp, b_r[None, :])

    loop = jnp.arange(N, dtype=jnp.int32)
    pad = jnp.zeros((EP - EACT,), jnp.int32)
    src = jnp.concatenate([edge_index[0], loop, pad])
    dst = jnp.concatenate([edge_index[1], loop, pad])

    h0, h1, den = _stage2(src, dst, xl0, xl1, xr, att)

    return _stage3(h0, h1, den[:, None], bias_gat[None, :], g_gat[None, :],
                   be_gat[None, :], W1.T, b1[None, :], g1[None, :],
                   be1[None, :], W2.T, b2[None, :], g2[None, :],
                   be2[None, :], W3.T, b3[None, :])
